# Initial kernel scaffold; baseline (speedup 1.0000x reference)
import jax
import jax.numpy as jnp
from jax.experimental import pallas as pl


def kernel(edge_index, dict_gc, adj_time, cur_time, hist_feat, hist_time, t2v_w, t2v_b, node_w, node_b, att_w, weight):
    raise NotImplementedError("write your pallas kernel here")


# speedup vs baseline: 1.5253x; 1.5253x over previous
import functools

import jax
import jax.numpy as jnp
from jax import lax
from jax.experimental import pallas as pl
from jax.experimental.pallas import tpu as pltpu
from jax.experimental.pallas import tpu_sc as plsc

N = 10000
E = 320000
D = 128
H = 128
WIN = 8

NC = 2
NS = 16
NW = NC * NS
EW = E // NW
C = 80
NCHUNK = EW // C
RPT = 624


def _node_encode_body(cur_ref, gc_ref, adj_ref, w_ref, b_ref, nw_ref, nb_ref,
                      aw_ref, nf_ref, proj_ref, ts_ref):
    t = cur_ref[0]
    adj = adj_ref[...]
    dt = jnp.abs(t - adj)
    wv = w_ref[...]
    bv = b_ref[...]
    per = jnp.cos(dt * wv + bv)
    lin = w_ref[0, 0] * dt + b_ref[0, 0]
    lane = lax.broadcasted_iota(jnp.int32, per.shape, 1)
    tv = jnp.where(lane == 0, lin, per)
    eff = gc_ref[...] * nw_ref[...] + nb_ref[...]
    nf0 = jnp.maximum(eff + tv, 0.0)
    nrm = jnp.sqrt(jnp.sum(nf0 * nf0, axis=1, keepdims=True))
    nf = nf0 / jnp.maximum(nrm, 1e-12)
    nf_ref[...] = nf
    proj_ref[...] = jnp.dot(nf, aw_ref[...], preferred_element_type=jnp.float32)
    ts_ref[...] = 1.0 / jnp.log(jnp.e + 2.0 * (t - adj))


def _node_encode(cur_time, dict_gc, adj_time, t2v_w, t2v_b, node_w, node_b, att_w):
    b = 1000
    grid = (N // b,)
    return pl.pallas_call(
        _node_encode_body,
        grid=grid,
        in_specs=[
            pl.BlockSpec(memory_space=pltpu.SMEM),
            pl.BlockSpec((b, 1), lambda i: (i, 0)),
            pl.BlockSpec((b, 1), lambda i: (i, 0)),
            pl.BlockSpec((1, D), lambda i: (0, 0)),
            pl.BlockSpec((1, D), lambda i: (0, 0)),
            pl.BlockSpec((1, D), lambda i: (0, 0)),
            pl.BlockSpec((1, D), lambda i: (0, 0)),
            pl.BlockSpec((D, D), lambda i: (0, 0)),
        ],
        out_specs=[
            pl.BlockSpec((b, D), lambda i: (i, 0)),
            pl.BlockSpec((b, D), lambda i: (i, 0)),
            pl.BlockSpec((b, 1), lambda i: (i, 0)),
        ],
        out_shape=[
            jax.ShapeDtypeStruct((N, D), jnp.float32),
            jax.ShapeDtypeStruct((N, D), jnp.float32),
            jax.ShapeDtypeStruct((N, 1), jnp.float32),
        ],
    )(cur_time, dict_gc.reshape(N, 1), adj_time.reshape(N, 1),
      t2v_w.reshape(1, D), t2v_b.reshape(1, D), node_w.reshape(1, D),
      node_b.reshape(1, D), att_w)


def _hist_body(cur_ref, ht_ref, hf_ref, fh_ref):
    t = cur_ref[0]
    w = 1.0 / (1.0 + 2.0 * (t - ht_ref[...]))
    fh_ref[...] = jnp.sum(w[..., None] * hf_ref[...], axis=1)


def _hist_reduce(cur_time, hist_time, hist_feat):
    b = 400
    grid = (N // b,)
    return pl.pallas_call(
        _hist_body,
        grid=grid,
        in_specs=[
            pl.BlockSpec(memory_space=pltpu.SMEM),
            pl.BlockSpec((b, WIN), lambda i: (i, 0)),
            pl.BlockSpec((b, WIN, 2 * D), lambda i: (i, 0, 0)),
        ],
        out_specs=pl.BlockSpec((b, 2 * D), lambda i: (i, 0)),
        out_shape=jax.ShapeDtypeStruct((N, 2 * D), jnp.float32),
    )(cur_time, hist_time, hist_feat)


def _sc_edge_body(nf_hbm, pj_hbm, ts_hbm, src_hbm, dst_hbm, zero_hbm, out_hbm,
                  srcv, dstv, nfr, pjr, msg, tst, acc, sem1, sem2):
    c = lax.axis_index("c")
    s = lax.axis_index("s")
    wid = c * NS + s

    pltpu.sync_copy(ts_hbm, tst)
    rbase = pl.multiple_of(s * RPT, 8)
    pltpu.sync_copy(zero_hbm.at[pl.ds(rbase, RPT)], acc.at[pl.ds(rbase, RPT)])

    @pl.when(s == NS - 1)
    def _():
        pltpu.sync_copy(zero_hbm.at[pl.ds(NS * RPT, N - NS * RPT)],
                        acc.at[pl.ds(NS * RPT, N - NS * RPT)])

    plsc.subcore_barrier()

    lanes = lax.iota(jnp.int32, 16)

    def chunk(g, carry):
        base = wid * EW + g * C
        pltpu.sync_copy(src_hbm.at[pl.ds(base, C)], srcv)
        pltpu.sync_copy(dst_hbm.at[pl.ds(base, C)], dstv)
        cp1 = pltpu.async_copy(nf_hbm.at[srcv], nfr, sem1)
        cp2 = pltpu.async_copy(pj_hbm.at[dstv], pjr, sem2)
        cp1.wait()
        cp2.wait()
        for i in range(C // 16):
            row16 = lanes + (i * 16)
            src16 = srcv[pl.ds(i * 16, 16)]
            ts16 = plsc.load_gather(tst, [src16])

            def dbody(dd, a):
                col = jnp.full((16,), dd, jnp.int32)
                x = plsc.load_gather(nfr, [row16, col])
                y = plsc.load_gather(pjr, [row16, col])
                return a + x * y

            att = lax.fori_loop(0, D, dbody, jnp.zeros((16,), jnp.float32),
                                unroll=8)
            sc = ts16 + att
            score = jnp.where(sc > 0.0, sc, 0.01 * sc)

            def mbody(dd, cc):
                col = jnp.full((16,), dd, jnp.int32)
                x = plsc.load_gather(nfr, [row16, col])
                plsc.store_scatter(msg, [row16, col], x * score)
                return cc

            lax.fori_loop(0, D, mbody, 0, unroll=8)
        pltpu.sync_copy(msg, acc.at[dstv], add=True)
        return carry

    lax.fori_loop(0, NCHUNK, chunk, 0)
    plsc.subcore_barrier()
    obase = pl.multiple_of(c * N + s * RPT, 8)
    pltpu.sync_copy(acc.at[pl.ds(rbase, RPT)], out_hbm.at[pl.ds(obase, RPT)])

    @pl.when(s == NS - 1)
    def _():
        pltpu.sync_copy(acc.at[pl.ds(NS * RPT, N - NS * RPT)],
                        out_hbm.at[pl.ds(c * N + NS * RPT, N - NS * RPT)])


def _sc_edge(nf, proj, ts, src, dst):
    mesh = plsc.VectorSubcoreMesh(core_axis_name="c", subcore_axis_name="s")
    zero = jnp.zeros((N, D), jnp.float32)
    fn = pl.kernel(
        _sc_edge_body,
        out_type=jax.ShapeDtypeStruct((NC * N, D), jnp.float32),
        mesh=mesh,
        scratch_types=[
            pltpu.VMEM((C,), jnp.int32),
            pltpu.VMEM((C,), jnp.int32),
            pltpu.VMEM((C, D), jnp.float32),
            pltpu.VMEM((C, D), jnp.float32),
            pltpu.VMEM((C, D), jnp.float32),
            pltpu.VMEM((N,), jnp.float32),
            pltpu.VMEM_SHARED((N, D), jnp.float32),
            pltpu.SemaphoreType.DMA,
            pltpu.SemaphoreType.DMA,
        ],
        compiler_params=pltpu.CompilerParams(needs_layout_passes=False),
    )
    return fn(nf, proj, ts, src, dst, zero)


def _combine_body(fh_ref, nf_ref, a0_ref, a1_ref, w1_ref, w2_ref, w3_ref, o_ref):
    neigh = a0_ref[...] + a1_ref[...]
    dn = (((1,), (1,)), ((), ()))
    acc = lax.dot_general(fh_ref[...], w1_ref[...], dn,
                          preferred_element_type=jnp.float32)
    acc += lax.dot_general(nf_ref[...], w2_ref[...], dn,
                           preferred_element_type=jnp.float32)
    acc += lax.dot_general(neigh, w3_ref[...], dn,
                           preferred_element_type=jnp.float32)
    o_ref[...] = jnp.maximum(acc, 0.0)


def _combine(fh, nf, a0, a1, weight):
    b = 1000
    grid = (N // b,)
    w1 = weight[:, : 2 * D]
    w2 = weight[:, 2 * D: 3 * D]
    w3 = weight[:, 3 * D:]
    return pl.pallas_call(
        _combine_body,
        grid=grid,
        in_specs=[
            pl.BlockSpec((b, 2 * D), lambda i: (i, 0)),
            pl.BlockSpec((b, D), lambda i: (i, 0)),
            pl.BlockSpec((b, D), lambda i: (i, 0)),
            pl.BlockSpec((b, D), lambda i: (i, 0)),
            pl.BlockSpec((H, 2 * D), lambda i: (0, 0)),
            pl.BlockSpec((H, D), lambda i: (0, 0)),
            pl.BlockSpec((H, D), lambda i: (0, 0)),
        ],
        out_specs=pl.BlockSpec((b, H), lambda i: (i, 0)),
        out_shape=jax.ShapeDtypeStruct((N, H), jnp.float32),
    )(fh, nf, a0, a1, w1, w2, w3)


def kernel(edge_index, dict_gc, adj_time, cur_time, hist_feat, hist_time,
           t2v_w, t2v_b, node_w, node_b, att_w, weight):
    src = edge_index[0].astype(jnp.int32)
    dst = edge_index[1].astype(jnp.int32)
    nf, proj, ts2 = _node_encode(cur_time, dict_gc, adj_time, t2v_w, t2v_b,
                                 node_w[:, 0], node_b, att_w)
    accs = _sc_edge(nf, proj, ts2.reshape(N), src, dst)
    fh = _hist_reduce(cur_time, hist_time, hist_feat)
    return _combine(fh, nf, accs[:N], accs[N:], weight)


# speedup vs baseline: 7.8498x; 5.1464x over previous
import functools

import jax
import jax.numpy as jnp
from jax import lax
from jax.experimental import pallas as pl
from jax.experimental.pallas import tpu as pltpu
from jax.experimental.pallas import tpu_sc as plsc

N = 10000
E = 320000
D = 128
H = 128
WIN = 8

NC = 2
NS = 16
NW = NC * NS
EW = E // NW
C = 80
NCHUNK = EW // C
RPT = 624
ROT = 1


def _node_encode_body(cur_ref, gc_ref, adj_ref, w_ref, b_ref, nw_ref, nb_ref,
                      aw_ref, nf_ref, proj_ref, ts_ref):
    t = cur_ref[0]
    adj = adj_ref[...]
    dt = jnp.abs(t - adj)
    wv = w_ref[...]
    bv = b_ref[...]
    per = jnp.cos(dt * wv + bv)
    lin = w_ref[0, 0] * dt + b_ref[0, 0]
    lane = lax.broadcasted_iota(jnp.int32, per.shape, 1)
    tv = jnp.where(lane == 0, lin, per)
    eff = gc_ref[...] * nw_ref[...] + nb_ref[...]
    nf0 = jnp.maximum(eff + tv, 0.0)
    nrm = jnp.sqrt(jnp.sum(nf0 * nf0, axis=1, keepdims=True))
    nf = nf0 / jnp.maximum(nrm, 1e-12)
    nf_ref[...] = nf
    proj_ref[...] = jnp.dot(nf, aw_ref[...], preferred_element_type=jnp.float32)
    ts_ref[...] = 1.0 / jnp.log(jnp.e + 2.0 * (t - adj))


def _node_encode(cur_time, dict_gc, adj_time, t2v_w, t2v_b, node_w, node_b, att_w):
    b = 1000
    grid = (N // b,)
    return pl.pallas_call(
        _node_encode_body,
        grid=grid,
        in_specs=[
            pl.BlockSpec(memory_space=pltpu.SMEM),
            pl.BlockSpec((b, 1), lambda i: (i, 0)),
            pl.BlockSpec((b, 1), lambda i: (i, 0)),
            pl.BlockSpec((1, D), lambda i: (0, 0)),
            pl.BlockSpec((1, D), lambda i: (0, 0)),
            pl.BlockSpec((1, D), lambda i: (0, 0)),
            pl.BlockSpec((1, D), lambda i: (0, 0)),
            pl.BlockSpec((D, D), lambda i: (0, 0)),
        ],
        out_specs=[
            pl.BlockSpec((b, D), lambda i: (i, 0)),
            pl.BlockSpec((b, D), lambda i: (i, 0)),
            pl.BlockSpec((b, 1), lambda i: (i, 0)),
        ],
        out_shape=[
            jax.ShapeDtypeStruct((N, D), jnp.float32),
            jax.ShapeDtypeStruct((N, D), jnp.float32),
            jax.ShapeDtypeStruct((N, 1), jnp.float32),
        ],
    )(cur_time, dict_gc.reshape(N, 1), adj_time.reshape(N, 1),
      t2v_w.reshape(1, D), t2v_b.reshape(1, D), node_w.reshape(1, D),
      node_b.reshape(1, D), att_w)


def _hist_body(cur_ref, ht_ref, hf_ref, fh_ref):
    t = cur_ref[0]
    w = 1.0 / (1.0 + 2.0 * (t - ht_ref[...]))
    fh_ref[...] = jnp.sum(w[..., None] * hf_ref[...], axis=1)


def _hist_reduce(cur_time, hist_time, hist_feat):
    b = 400
    grid = (N // b,)
    return pl.pallas_call(
        _hist_body,
        grid=grid,
        in_specs=[
            pl.BlockSpec(memory_space=pltpu.SMEM),
            pl.BlockSpec((b, WIN), lambda i: (i, 0)),
            pl.BlockSpec((b, WIN, 2 * D), lambda i: (i, 0, 0)),
        ],
        out_specs=pl.BlockSpec((b, 2 * D), lambda i: (i, 0)),
        out_shape=jax.ShapeDtypeStruct((N, 2 * D), jnp.float32),
    )(cur_time, hist_time, hist_feat)


def _sc_edge_body(nf_hbm, pj_hbm, ts_hbm, src_hbm, dst_hbm, zero_hbm, out_hbm,
                  srcv, dstv, nfr, pjr, msg, tst, acc, sem1, sem2):
    c = lax.axis_index("c")
    s = lax.axis_index("s")
    wid = c * NS + s

    pltpu.sync_copy(ts_hbm, tst)
    rbase = pl.multiple_of(s * RPT, 8)
    pltpu.sync_copy(zero_hbm.at[pl.ds(rbase, RPT)], acc.at[pl.ds(rbase, RPT)])

    @pl.when(s == NS - 1)
    def _():
        pltpu.sync_copy(zero_hbm.at[pl.ds(NS * RPT, N - NS * RPT)],
                        acc.at[pl.ds(NS * RPT, N - NS * RPT)])

    plsc.subcore_barrier()

    lanes = lax.iota(jnp.int32, 16)

    def chunk(g, carry):
        base = wid * EW + g * C
        pltpu.sync_copy(src_hbm.at[pl.ds(base, C)], srcv)
        pltpu.sync_copy(dst_hbm.at[pl.ds(base, C)], dstv)
        cp1 = pltpu.async_copy(nf_hbm.at[srcv], nfr, sem1)
        cp2 = pltpu.async_copy(pj_hbm.at[dstv], pjr, sem2)
        cp1.wait()
        cp2.wait()
        for i in range(C // 16):
            row16 = lanes + (i * 16)
            src16 = srcv[pl.ds(i * 16, 16)]
            ts16 = plsc.load_gather(tst, [src16])
            rot = lanes * ROT

            def dbody(dd, a):
                col = jnp.bitwise_and(rot + dd, D - 1)
                x = plsc.load_gather(nfr, [row16, col])
                y = plsc.load_gather(pjr, [row16, col])
                return a + x * y

            att = plsc.parallel_loop(
                0, D, unroll=8, carry=jnp.zeros((16,), jnp.float32))(dbody)
            sc = ts16 + att
            score = jnp.where(sc > 0.0, sc, 0.01 * sc)

            def mbody(dd):
                col = jnp.bitwise_and(rot + dd, D - 1)
                x = plsc.load_gather(nfr, [row16, col])
                plsc.store_scatter(msg, [row16, col], x * score)

            plsc.parallel_loop(0, D, unroll=8)(mbody)
        pltpu.sync_copy(msg, acc.at[dstv], add=True)
        return carry

    lax.fori_loop(0, NCHUNK, chunk, 0)
    plsc.subcore_barrier()
    obase = pl.multiple_of(c * N + s * RPT, 8)
    pltpu.sync_copy(acc.at[pl.ds(rbase, RPT)], out_hbm.at[pl.ds(obase, RPT)])

    @pl.when(s == NS - 1)
    def _():
        pltpu.sync_copy(acc.at[pl.ds(NS * RPT, N - NS * RPT)],
                        out_hbm.at[pl.ds(c * N + NS * RPT, N - NS * RPT)])


def _sc_edge(nf, proj, ts, src, dst):
    mesh = plsc.VectorSubcoreMesh(core_axis_name="c", subcore_axis_name="s")
    zero = jnp.zeros((N, D), jnp.float32)
    fn = pl.kernel(
        _sc_edge_body,
        out_type=jax.ShapeDtypeStruct((NC * N, D), jnp.float32),
        mesh=mesh,
        scratch_types=[
            pltpu.VMEM((C,), jnp.int32),
            pltpu.VMEM((C,), jnp.int32),
            pltpu.VMEM((C, D), jnp.float32),
            pltpu.VMEM((C, D), jnp.float32),
            pltpu.VMEM((C, D), jnp.float32),
            pltpu.VMEM((N,), jnp.float32),
            pltpu.VMEM_SHARED((N, D), jnp.float32),
            pltpu.SemaphoreType.DMA,
            pltpu.SemaphoreType.DMA,
        ],
        compiler_params=pltpu.CompilerParams(needs_layout_passes=False),
    )
    return fn(nf, proj, ts, src, dst, zero)


def _combine_body(fh_ref, nf_ref, a0_ref, a1_ref, w1_ref, w2_ref, w3_ref, o_ref):
    neigh = a0_ref[...] + a1_ref[...]
    dn = (((1,), (1,)), ((), ()))
    acc = lax.dot_general(fh_ref[...], w1_ref[...], dn,
                          preferred_element_type=jnp.float32)
    acc += lax.dot_general(nf_ref[...], w2_ref[...], dn,
                           preferred_element_type=jnp.float32)
    acc += lax.dot_general(neigh, w3_ref[...], dn,
                           preferred_element_type=jnp.float32)
    o_ref[...] = jnp.maximum(acc, 0.0)


def _combine(fh, nf, a0, a1, weight):
    b = 1000
    grid = (N // b,)
    w1 = weight[:, : 2 * D]
    w2 = weight[:, 2 * D: 3 * D]
    w3 = weight[:, 3 * D:]
    return pl.pallas_call(
        _combine_body,
        grid=grid,
        in_specs=[
            pl.BlockSpec((b, 2 * D), lambda i: (i, 0)),
            pl.BlockSpec((b, D), lambda i: (i, 0)),
            pl.BlockSpec((b, D), lambda i: (i, 0)),
            pl.BlockSpec((b, D), lambda i: (i, 0)),
            pl.BlockSpec((H, 2 * D), lambda i: (0, 0)),
            pl.BlockSpec((H, D), lambda i: (0, 0)),
            pl.BlockSpec((H, D), lambda i: (0, 0)),
        ],
        out_specs=pl.BlockSpec((b, H), lambda i: (i, 0)),
        out_shape=jax.ShapeDtypeStruct((N, H), jnp.float32),
    )(fh, nf, a0, a1, w1, w2, w3)


def kernel(edge_index, dict_gc, adj_time, cur_time, hist_feat, hist_time,
           t2v_w, t2v_b, node_w, node_b, att_w, weight):
    src = edge_index[0].astype(jnp.int32)
    dst = edge_index[1].astype(jnp.int32)
    nf, proj, ts2 = _node_encode(cur_time, dict_gc, adj_time, t2v_w, t2v_b,
                                 node_w[:, 0], node_b, att_w)
    accs = _sc_edge(nf, proj, ts2.reshape(N), src, dst)
    fh = _hist_reduce(cur_time, hist_time, hist_feat)
    return _combine(fh, nf, accs[:N], accs[N:], weight)
